# Initial kernel scaffold; baseline (speedup 1.0000x reference)
import jax
import jax.numpy as jnp
from jax.experimental import pallas as pl


def kernel(x, weight_param):
    raise NotImplementedError("write your pallas kernel here")


# speedup vs baseline: 3.5992x; 3.5992x over previous
import functools

import jax
import jax.numpy as jnp
import numpy as np
from jax.experimental import pallas as pl

_ITEM_Z = np.array(
    [1, 1, 1, 0, 1, 1, 1, 1, 1, 1, 0, 1, 1, 1, 1, 1,
     1, 0, 1, 1, 1, 1, 1, 1, 0, 1, 1, 1, 1, 1, 1, 1],
    dtype=np.int64,
)
_INPUTS_PER_ITEM = 64
_N_ITEMS = _ITEM_Z.size
_OUT_FEATURES = _N_ITEMS * _INPUTS_PER_ITEM
_N_ACTIVE = int(_ITEM_Z.sum())
_ZERO_OUT_IDX = np.array([0, 63, 100, 511, 1024, 1500, 2047], dtype=np.int64)

_E = np.zeros((_N_ACTIVE, _OUT_FEATURES), dtype=np.float32)
_rank = 0
for _i in range(_N_ITEMS):
    if _ITEM_Z[_i]:
        _E[_rank, _i * _INPUTS_PER_ITEM:(_i + 1) * _INPUTS_PER_ITEM] = 1.0
        _rank += 1
_E[:, _ZERO_OUT_IDX] = 0.0

_ROW_BLOCK = 1024


def _mul_body(wp_ref, e_ref, x_ref, o_ref):
    w = jnp.dot(wp_ref[...], e_ref[...], preferred_element_type=jnp.float32)
    o_ref[...] = x_ref[...] * w


@jax.jit
def kernel(x, weight_param):
    batch, feats = x.shape
    wp = weight_param.reshape(1, _N_ACTIVE)
    e = jnp.asarray(_E)
    grid = (batch // _ROW_BLOCK,)
    return pl.pallas_call(
        _mul_body,
        grid=grid,
        in_specs=[
            pl.BlockSpec((1, _N_ACTIVE), lambda i: (0, 0)),
            pl.BlockSpec((_N_ACTIVE, feats), lambda i: (0, 0)),
            pl.BlockSpec((_ROW_BLOCK, feats), lambda i: (i, 0)),
        ],
        out_specs=pl.BlockSpec((_ROW_BLOCK, feats), lambda i: (i, 0)),
        out_shape=jax.ShapeDtypeStruct((batch, feats), x.dtype),
    )(wp, e, x)
